# baseline (speedup 1.0000x reference)
import functools

import jax
import jax.numpy as jnp
from jax import lax
from jax.experimental import pallas as pl
from jax.experimental.pallas import tpu as pltpu
from jax.experimental.pallas import tpu_sc as plsc

B = 8
L = 1024
M = L - 2
ROWS = M - 1
PAD0 = 8


def _fill_pad(pad_v, fq_v, e_v, f_v, rev):
    one = jnp.ones((16,), jnp.float32)
    for k in range(65):
        pad_v[pl.ds(k * 16, 16)] = one
    for k in range(64):
        if not rev:
            ev = e_v[pl.ds(9 + k * 16, 16)]
            fv = f_v[pl.ds(9 + k * 16, 16)]
        else:
            ev = lax.rev(e_v[pl.ds(1015 - k * 16, 16)], (0,))
            fv = lax.rev(f_v[pl.ds(1015 - k * 16, 16)], (0,))
        pad_v[pl.ds(PAD0 + M + k * 16, 16)] = ev
        fq_v[pl.ds(k * 16, 16)] = fv
    pad_v[pl.ds(PAD0 + M + 1022, 16)] = one
    fq_v[pl.ds(M, 16)] = one


def _body(score_hbm, sidx_hbm, fwd_hbm, bwd_hbm,
          score_v, idx_v, row_v, e_v, f_v, pad_v, fq_v, run_v,
          ob0, ob1, sem0, sem1):
    c = lax.axis_index("c")
    s = lax.axis_index("s")
    b = jnp.where(s < 8, s, s - 8)

    pltpu.sync_copy(score_hbm, score_v)
    pltpu.sync_copy(sidx_hbm.at[b], idx_v)
    for k in range(64):
        iv = idx_v[pl.ds(k * 16, 16)]
        row_v[pl.ds(8 + k * 16, 16)] = plsc.load_gather(score_v, [iv])

    for k in range(64):
        rv = row_v[pl.ds(8 + k * 16, 16)]
        e_v[pl.ds(8 + k * 16, 16)] = jnp.exp(
            jnp.clip(-5.0 * rv, -85.0, 85.0))
        f_v[pl.ds(8 + k * 16, 16)] = jnp.exp(
            jnp.clip(5.0 * rv - 1.25, -85.0, 85.0))

    def compute_chunk(t, s0, ob):
        for g in range(8):
            c0g = s0 + g * 16
            fj = fq_v[pl.ds(c0g, 16)]

            def fac(off, fj=fj):
                gg = pad_v[pl.ds(off, 16)] * fj
                g2 = gg * gg
                return 1.0 / (1.0 + g2 * g2)

            def blockfn(sb, run, g=g, c0g=c0g, fac=fac, t=t):
                i0 = sb * 8
                offb = ROWS + PAD0 + c0g - i0
                gs = [fac(offb - u) for u in range(8)]
                a1 = gs[0] * gs[1]
                a3 = gs[2] * gs[3]
                a5 = gs[4] * gs[5]
                a7 = gs[6] * gs[7]
                b3 = a1 * a3
                p = [gs[0], a1, a1 * gs[2], b3, b3 * gs[4], b3 * a5,
                     b3 * (a5 * gs[6]), b3 * (a5 * a7)]
                r0 = i0 - t * 64
                for u in range(8):
                    ob[r0 + u, pl.ds(g * 16, 16)] = run * p[u]
                return run * p[7]

            f2 = fj * fj
            v1 = 1.0 / (1.0 + f2 * f2)
            v2 = v1 * v1
            v4 = v2 * v2
            pw = [v1, v2, v2 * v1, v4, v4 * v1, v4 * v2, v4 * (v2 * v1),
                  v4 * v4]

            def blockfn_c(sb, run, g=g, pw=pw, t=t):
                r0 = sb * 8 - t * 64
                for u in range(8):
                    ob[r0 + u, pl.ds(g * 16, 16)] = run * pw[u]
                return run * pw[7]

            ta = jnp.minimum((c0g + 22) // 8, 128)
            lo = t * 8
            hi = lo + 8
            a = jnp.clip(ta, lo, hi)
            run = run_v[pl.ds(g * 16, 16)]
            run = lax.fori_loop(lo, a, blockfn, run)
            run = lax.fori_loop(a, hi, blockfn_c, run)
            run_v[pl.ds(g * 16, 16)] = run

    def run_direction(out_hbm, rev):
        _fill_pad(pad_v, fq_v, e_v, f_v, rev=rev)
        one = jnp.ones((16,), jnp.float32)

        def dma_start(ob, t, s0, sem):
            pltpu.async_copy(
                ob, out_hbm.at[b, pl.ds(t * 64, 64), pl.ds(s0, 128)], sem)

        def dma_drain(ob, sem):
            pltpu.make_async_copy(
                ob, out_hbm.at[b, pl.ds(0, 64), pl.ds(0, 128)], sem).wait()

        def span_body(sp, carry):
            s0 = (2 * sp + c) * 128
            for g in range(8):
                run_v[pl.ds(g * 16, 16)] = one

            def wbody(w, carry2):
                t0 = 2 * w

                @pl.when(w > 0)
                def _():
                    dma_drain(ob0, sem0)

                compute_chunk(t0, s0, ob0)
                dma_start(ob0, t0, s0, sem0)

                @pl.when(w > 0)
                def _():
                    dma_drain(ob1, sem1)

                compute_chunk(t0 + 1, s0, ob1)
                dma_start(ob1, t0 + 1, s0, sem1)
                return carry2

            lax.fori_loop(0, 8, wbody, jnp.int32(0))
            dma_drain(ob0, sem0)
            dma_drain(ob1, sem1)
            return carry

        lax.fori_loop(0, 4, span_body, jnp.int32(0))

    @pl.when(s < 8)
    def _():
        run_direction(fwd_hbm, rev=False)

    @pl.when(s >= 8)
    def _():
        run_direction(bwd_hbm, rev=True)


@functools.partial(jax.jit, static_argnames=())
def _gate_net(score, score_idx):
    mesh = plsc.VectorSubcoreMesh(core_axis_name="c", subcore_axis_name="s")
    out_ty = (jax.ShapeDtypeStruct((B, ROWS, M), jnp.float32),
              jax.ShapeDtypeStruct((B, ROWS, M), jnp.float32))
    fn = pl.kernel(
        _body,
        mesh=mesh,
        out_type=out_ty,
        scratch_types=[
            pltpu.VMEM((B * L,), jnp.float32),
            pltpu.VMEM((L,), jnp.int32),
            pltpu.VMEM((8 + L + 8,), jnp.float32),
            pltpu.VMEM((8 + L + 8,), jnp.float32),
            pltpu.VMEM((8 + L + 8,), jnp.float32),
            pltpu.VMEM((2 * L + 32,), jnp.float32),
            pltpu.VMEM((L + 16,), jnp.float32),
            pltpu.VMEM((128,), jnp.float32),
            pltpu.VMEM((64, 128), jnp.float32),
            pltpu.VMEM((64, 128), jnp.float32),
            pltpu.SemaphoreType.DMA,
            pltpu.SemaphoreType.DMA,
        ],
        compiler_params=pltpu.CompilerParams(needs_layout_passes=False),
    )
    return fn(score, score_idx)


def kernel(score, rep_srcs, rep_idx, score_idx):
    del rep_srcs, rep_idx
    return _gate_net(score, score_idx.astype(jnp.int32))


# speedup vs baseline: 1.2855x; 1.2855x over previous
import functools

import jax
import jax.numpy as jnp
from jax import lax
from jax.experimental import pallas as pl
from jax.experimental.pallas import tpu as pltpu
from jax.experimental.pallas import tpu_sc as plsc

B = 8
L = 1024
M = L - 2
ROWS = M - 1
PAD0 = 8


def _fill_pad(pad_v, fq_v, e_v, f_v, rev):
    one = jnp.ones((16,), jnp.float32)
    for k in range(65):
        pad_v[pl.ds(k * 16, 16)] = one
    for k in range(64):
        if not rev:
            ev = e_v[pl.ds(9 + k * 16, 16)]
            fv = f_v[pl.ds(9 + k * 16, 16)]
        else:
            ev = lax.rev(e_v[pl.ds(1015 - k * 16, 16)], (0,))
            fv = lax.rev(f_v[pl.ds(1015 - k * 16, 16)], (0,))
        pad_v[pl.ds(PAD0 + M + k * 16, 16)] = ev
        fq_v[pl.ds(k * 16, 16)] = fv
    pad_v[pl.ds(PAD0 + M + 1022, 16)] = one
    fq_v[pl.ds(M, 16)] = one


def _body(score_hbm, sidx_hbm, fwd_hbm, bwd_hbm,
          score_v, idx_v, row_v, e_v, f_v, pad_v, fq_v, run_v,
          ob0, ob1, sem0, sem1):
    c = lax.axis_index("c")
    s = lax.axis_index("s")
    b = jnp.where(s < 8, s, s - 8)

    pltpu.sync_copy(score_hbm, score_v)
    pltpu.sync_copy(sidx_hbm.at[b], idx_v)
    for k in range(64):
        iv = idx_v[pl.ds(k * 16, 16)]
        row_v[pl.ds(8 + k * 16, 16)] = plsc.load_gather(score_v, [iv])

    for k in range(64):
        rv = row_v[pl.ds(8 + k * 16, 16)]
        e_v[pl.ds(8 + k * 16, 16)] = jnp.exp(
            jnp.clip(-5.0 * rv, -85.0, 85.0))
        f_v[pl.ds(8 + k * 16, 16)] = jnp.exp(
            jnp.clip(5.0 * rv - 1.25, -85.0, 85.0))

    def compute_chunk(t, s0, ob):
        for g in range(8):
            c0g = s0 + g * 16
            fj = fq_v[pl.ds(c0g, 16)]

            def fac(off, fj=fj):
                gg = pad_v[pl.ds(off, 16)] * fj
                g2 = gg * gg
                return 1.0 / (1.0 + g2 * g2)

            def blockfn(sb, run, g=g, c0g=c0g, fac=fac, t=t):
                i0 = sb * 8
                offb = ROWS + PAD0 + c0g - i0
                gs = [fac(offb - u) for u in range(8)]
                a1 = gs[0] * gs[1]
                a3 = gs[2] * gs[3]
                a5 = gs[4] * gs[5]
                a7 = gs[6] * gs[7]
                b3 = a1 * a3
                p = [gs[0], a1, a1 * gs[2], b3, b3 * gs[4], b3 * a5,
                     b3 * (a5 * gs[6]), b3 * (a5 * a7)]
                r0 = i0 - t * 128
                for u in range(8):
                    ob[r0 + u, pl.ds(g * 16, 16)] = run * p[u]
                return run * p[7]

            f2 = fj * fj
            v1 = 1.0 / (1.0 + f2 * f2)
            v2 = v1 * v1
            v4 = v2 * v2
            pw = [v1, v2, v2 * v1, v4, v4 * v1, v4 * v2, v4 * (v2 * v1),
                  v4 * v4]

            def blockfn_c(sb, run, g=g, pw=pw, t=t):
                r0 = sb * 8 - t * 128
                for u in range(8):
                    ob[r0 + u, pl.ds(g * 16, 16)] = run * pw[u]
                return run * pw[7]

            ta = jnp.minimum((c0g + 22) // 8, 128)
            lo = t * 16
            hi = lo + 16
            a = jnp.clip(ta, lo, hi)
            run = run_v[pl.ds(g * 16, 16)]
            run = lax.fori_loop(lo, a, blockfn, run)
            run = lax.fori_loop(a, hi, blockfn_c, run)
            run_v[pl.ds(g * 16, 16)] = run

    def run_direction(out_hbm, rev):
        _fill_pad(pad_v, fq_v, e_v, f_v, rev=rev)
        one = jnp.ones((16,), jnp.float32)

        def dma_start(ob, t, s0, sem):
            pltpu.async_copy(
                ob, out_hbm.at[b, pl.ds(t * 128, 128), pl.ds(s0, 128)], sem)

        def dma_drain(ob, sem):
            pltpu.make_async_copy(
                ob, out_hbm.at[b, pl.ds(0, 128), pl.ds(0, 128)], sem).wait()

        def span_body(sp, carry):
            s0 = (2 * sp + c) * 128
            for g in range(8):
                run_v[pl.ds(g * 16, 16)] = one

            def wbody(w, carry2):
                t0 = 2 * w

                @pl.when(w > 0)
                def _():
                    dma_drain(ob0, sem0)

                compute_chunk(t0, s0, ob0)
                dma_start(ob0, t0, s0, sem0)

                @pl.when(w > 0)
                def _():
                    dma_drain(ob1, sem1)

                compute_chunk(t0 + 1, s0, ob1)
                dma_start(ob1, t0 + 1, s0, sem1)
                return carry2

            lax.fori_loop(0, 4, wbody, jnp.int32(0))
            dma_drain(ob0, sem0)
            dma_drain(ob1, sem1)
            return carry

        lax.fori_loop(0, 4, span_body, jnp.int32(0))

    @pl.when(s < 8)
    def _():
        run_direction(fwd_hbm, rev=False)

    @pl.when(s >= 8)
    def _():
        run_direction(bwd_hbm, rev=True)


@functools.partial(jax.jit, static_argnames=())
def _gate_net(score, score_idx):
    mesh = plsc.VectorSubcoreMesh(core_axis_name="c", subcore_axis_name="s")
    out_ty = (jax.ShapeDtypeStruct((B, ROWS, M), jnp.float32),
              jax.ShapeDtypeStruct((B, ROWS, M), jnp.float32))
    fn = pl.kernel(
        _body,
        mesh=mesh,
        out_type=out_ty,
        scratch_types=[
            pltpu.VMEM((B * L,), jnp.float32),
            pltpu.VMEM((L,), jnp.int32),
            pltpu.VMEM((8 + L + 8,), jnp.float32),
            pltpu.VMEM((8 + L + 8,), jnp.float32),
            pltpu.VMEM((8 + L + 8,), jnp.float32),
            pltpu.VMEM((2 * L + 32,), jnp.float32),
            pltpu.VMEM((L + 16,), jnp.float32),
            pltpu.VMEM((128,), jnp.float32),
            pltpu.VMEM((128, 128), jnp.float32),
            pltpu.VMEM((128, 128), jnp.float32),
            pltpu.SemaphoreType.DMA,
            pltpu.SemaphoreType.DMA,
        ],
        compiler_params=pltpu.CompilerParams(needs_layout_passes=False),
    )
    return fn(score, score_idx)


def kernel(score, rep_srcs, rep_idx, score_idx):
    del rep_srcs, rep_idx
    return _gate_net(score, score_idx.astype(jnp.int32))


# speedup vs baseline: 1.3652x; 1.0620x over previous
import functools

import jax
import jax.numpy as jnp
from jax import lax
from jax.experimental import pallas as pl
from jax.experimental.pallas import tpu as pltpu
from jax.experimental.pallas import tpu_sc as plsc

B = 8
L = 1024
M = L - 2
ROWS = M - 1
PAD0 = 8


def _fill_pad(pad_v, fq_v, e_v, f_v, rev):
    one = jnp.ones((16,), jnp.float32)
    for k in range(65):
        pad_v[pl.ds(k * 16, 16)] = one
    for k in range(64):
        if not rev:
            ev = e_v[pl.ds(9 + k * 16, 16)]
            fv = f_v[pl.ds(9 + k * 16, 16)]
        else:
            ev = lax.rev(e_v[pl.ds(1015 - k * 16, 16)], (0,))
            fv = lax.rev(f_v[pl.ds(1015 - k * 16, 16)], (0,))
        pad_v[pl.ds(PAD0 + M + k * 16, 16)] = ev
        fq_v[pl.ds(k * 16, 16)] = fv
    pad_v[pl.ds(PAD0 + M + 1022, 16)] = one
    fq_v[pl.ds(M, 16)] = one


def _body(score_hbm, sidx_hbm, fwd_hbm, bwd_hbm,
          score_v, idx_v, row_v, e_v, f_v, pad_v, fq_v, run_v,
          ob0, ob1, sem0, sem1):
    c = lax.axis_index("c")
    s = lax.axis_index("s")
    b = jnp.where(s < 8, s, s - 8)

    pltpu.sync_copy(score_hbm, score_v)
    pltpu.sync_copy(sidx_hbm.at[b], idx_v)
    for k in range(64):
        iv = idx_v[pl.ds(k * 16, 16)]
        row_v[pl.ds(8 + k * 16, 16)] = plsc.load_gather(score_v, [iv])

    for k in range(64):
        rv = row_v[pl.ds(8 + k * 16, 16)]
        e_v[pl.ds(8 + k * 16, 16)] = jnp.exp(
            jnp.clip(-5.0 * rv, -85.0, 85.0))
        f_v[pl.ds(8 + k * 16, 16)] = jnp.exp(
            jnp.clip(5.0 * rv - 1.25, -85.0, 85.0))

    def compute_chunk(t, s0, ob):
        for g in range(8):
            c0g = s0 + g * 16
            fj = fq_v[pl.ds(c0g, 16)]

            def fac(off, fj=fj):
                gg = pad_v[pl.ds(off, 16)] * fj
                g2 = gg * gg
                return 1.0 / (1.0 + g2 * g2)

            def blockfn(sb, run, g=g, c0g=c0g, fac=fac, t=t):
                i0 = sb * 8
                offb = ROWS + PAD0 + c0g - i0
                gs = [fac(offb - u) for u in range(8)]
                a1 = gs[0] * gs[1]
                a3 = gs[2] * gs[3]
                a5 = gs[4] * gs[5]
                a7 = gs[6] * gs[7]
                b3 = a1 * a3
                p = [gs[0], a1, a1 * gs[2], b3, b3 * gs[4], b3 * a5,
                     b3 * (a5 * gs[6]), b3 * (a5 * a7)]
                r0 = i0 - t * 256
                for u in range(8):
                    ob[r0 + u, pl.ds(g * 16, 16)] = run * p[u]
                return run * p[7]

            f2 = fj * fj
            v1 = 1.0 / (1.0 + f2 * f2)
            v2 = v1 * v1
            v4 = v2 * v2
            pw = [v1, v2, v2 * v1, v4, v4 * v1, v4 * v2, v4 * (v2 * v1),
                  v4 * v4]

            def blockfn_c(sb, run, g=g, pw=pw, t=t):
                r0 = sb * 8 - t * 256
                for u in range(8):
                    ob[r0 + u, pl.ds(g * 16, 16)] = run * pw[u]
                return run * pw[7]

            ta = jnp.minimum((c0g + 22) // 8, 128)
            lo = t * 32
            hi = lo + 32
            a = jnp.clip(ta, lo, hi)
            run = run_v[pl.ds(g * 16, 16)]
            run = lax.fori_loop(lo, a, blockfn, run)
            run = lax.fori_loop(a, hi, blockfn_c, run)
            run_v[pl.ds(g * 16, 16)] = run

    def run_direction(out_hbm, rev):
        _fill_pad(pad_v, fq_v, e_v, f_v, rev=rev)
        one = jnp.ones((16,), jnp.float32)

        def dma_start(ob, t, s0, sem):
            pltpu.async_copy(
                ob, out_hbm.at[b, pl.ds(t * 256, 256), pl.ds(s0, 128)], sem)

        def dma_drain(ob, sem):
            pltpu.make_async_copy(
                ob, out_hbm.at[b, pl.ds(0, 256), pl.ds(0, 128)], sem).wait()

        def span_body(sp, carry):
            s0 = (2 * sp + c) * 128
            for g in range(8):
                run_v[pl.ds(g * 16, 16)] = one

            def wbody(w, carry2):
                t0 = 2 * w

                @pl.when(w > 0)
                def _():
                    dma_drain(ob0, sem0)

                compute_chunk(t0, s0, ob0)
                dma_start(ob0, t0, s0, sem0)

                @pl.when(w > 0)
                def _():
                    dma_drain(ob1, sem1)

                compute_chunk(t0 + 1, s0, ob1)
                dma_start(ob1, t0 + 1, s0, sem1)
                return carry2

            lax.fori_loop(0, 2, wbody, jnp.int32(0))
            dma_drain(ob0, sem0)
            dma_drain(ob1, sem1)
            return carry

        lax.fori_loop(0, 4, span_body, jnp.int32(0))

    @pl.when(s < 8)
    def _():
        run_direction(fwd_hbm, rev=False)

    @pl.when(s >= 8)
    def _():
        run_direction(bwd_hbm, rev=True)


@functools.partial(jax.jit, static_argnames=())
def _gate_net(score, score_idx):
    mesh = plsc.VectorSubcoreMesh(core_axis_name="c", subcore_axis_name="s")
    out_ty = (jax.ShapeDtypeStruct((B, ROWS, M), jnp.float32),
              jax.ShapeDtypeStruct((B, ROWS, M), jnp.float32))
    fn = pl.kernel(
        _body,
        mesh=mesh,
        out_type=out_ty,
        scratch_types=[
            pltpu.VMEM((B * L,), jnp.float32),
            pltpu.VMEM((L,), jnp.int32),
            pltpu.VMEM((8 + L + 8,), jnp.float32),
            pltpu.VMEM((8 + L + 8,), jnp.float32),
            pltpu.VMEM((8 + L + 8,), jnp.float32),
            pltpu.VMEM((2 * L + 32,), jnp.float32),
            pltpu.VMEM((L + 16,), jnp.float32),
            pltpu.VMEM((128,), jnp.float32),
            pltpu.VMEM((256, 128), jnp.float32),
            pltpu.VMEM((256, 128), jnp.float32),
            pltpu.SemaphoreType.DMA,
            pltpu.SemaphoreType.DMA,
        ],
        compiler_params=pltpu.CompilerParams(needs_layout_passes=False),
    )
    return fn(score, score_idx)


def kernel(score, rep_srcs, rep_idx, score_idx):
    del rep_srcs, rep_idx
    return _gate_net(score, score_idx.astype(jnp.int32))


# speedup vs baseline: 1.3969x; 1.0232x over previous
import functools

import jax
import jax.numpy as jnp
from jax import lax
from jax.experimental import pallas as pl
from jax.experimental.pallas import tpu as pltpu
from jax.experimental.pallas import tpu_sc as plsc

B = 8
L = 1024
M = L - 2
ROWS = M - 1
PAD0 = 8


def _fill_pad(pad_v, fq_v, e_v, f_v, rev):
    one = jnp.ones((16,), jnp.float32)
    for k in range(65):
        pad_v[pl.ds(k * 16, 16)] = one
    for k in range(64):
        if not rev:
            ev = e_v[pl.ds(9 + k * 16, 16)]
            fv = f_v[pl.ds(9 + k * 16, 16)]
        else:
            ev = lax.rev(e_v[pl.ds(1015 - k * 16, 16)], (0,))
            fv = lax.rev(f_v[pl.ds(1015 - k * 16, 16)], (0,))
        pad_v[pl.ds(PAD0 + M + k * 16, 16)] = ev
        fq_v[pl.ds(k * 16, 16)] = fv
    pad_v[pl.ds(PAD0 + M + 1022, 16)] = one
    fq_v[pl.ds(M, 16)] = one


def _body(score_hbm, sidx_hbm, fwd_hbm, bwd_hbm,
          score_v, idx_v, row_v, e_v, f_v, pad_v, fq_v, run_v,
          ob0, ob1, sem0, sem1):
    c = lax.axis_index("c")
    s = lax.axis_index("s")
    b = jnp.where(s < 8, s, s - 8)

    pltpu.sync_copy(score_hbm, score_v)
    pltpu.sync_copy(sidx_hbm.at[b], idx_v)
    for k in range(64):
        iv = idx_v[pl.ds(k * 16, 16)]
        row_v[pl.ds(8 + k * 16, 16)] = plsc.load_gather(score_v, [iv])

    for k in range(64):
        rv = row_v[pl.ds(8 + k * 16, 16)]
        e_v[pl.ds(8 + k * 16, 16)] = jnp.exp(
            jnp.clip(-5.0 * rv, -85.0, 85.0))
        f_v[pl.ds(8 + k * 16, 16)] = jnp.exp(
            jnp.clip(5.0 * rv - 1.25, -85.0, 85.0))

    def compute_chunk(t, s0, ob):
        for g in range(8):
            c0g = s0 + g * 16
            fj = fq_v[pl.ds(c0g, 16)]

            def fac(off, fj=fj):
                gg = pad_v[pl.ds(off, 16)] * fj
                g2 = gg * gg
                return 1.0 / (1.0 + g2 * g2)

            def blockfn(sb, run, g=g, c0g=c0g, fac=fac, t=t):
                i0 = sb * 8
                offb = ROWS + PAD0 + c0g - i0
                gs = [fac(offb - u) for u in range(8)]
                a1 = gs[0] * gs[1]
                a3 = gs[2] * gs[3]
                a5 = gs[4] * gs[5]
                a7 = gs[6] * gs[7]
                b3 = a1 * a3
                p = [gs[0], a1, a1 * gs[2], b3, b3 * gs[4], b3 * a5,
                     b3 * (a5 * gs[6]), b3 * (a5 * a7)]
                r0 = i0 - t * 256
                for u in range(8):
                    ob[r0 + u, pl.ds(g * 16, 16)] = run * p[u]
                return run * p[7]

            f2 = fj * fj
            v1 = 1.0 / (1.0 + f2 * f2)
            v2 = v1 * v1
            v4 = v2 * v2
            pw = [v1, v2, v2 * v1, v4, v4 * v1, v4 * v2, v4 * (v2 * v1),
                  v4 * v4]

            def blockfn_c(sb, run, g=g, pw=pw, t=t):
                r0 = sb * 8 - t * 256
                for u in range(8):
                    ob[r0 + u, pl.ds(g * 16, 16)] = run * pw[u]
                return run * pw[7]

            ta = jnp.minimum((c0g + 22) // 8, 128)
            lo = t * 32
            hi = lo + 32
            a = jnp.clip(ta, lo, hi)
            run = run_v[pl.ds(g * 16, 16)]
            run = lax.fori_loop(lo, a, blockfn, run)
            run = lax.fori_loop(a, hi, blockfn_c, run)
            run_v[pl.ds(g * 16, 16)] = run

    def run_direction(out_hbm, rev):
        _fill_pad(pad_v, fq_v, e_v, f_v, rev=rev)
        one = jnp.ones((16,), jnp.float32)

        def dma_start(ob, t, s0, sem):
            pltpu.async_copy(
                ob, out_hbm.at[b, pl.ds(t * 256, 256), pl.ds(s0, 128)], sem)

        def dma_drain(ob, sem):
            pltpu.make_async_copy(
                ob, out_hbm.at[b, pl.ds(0, 256), pl.ds(0, 128)], sem).wait()

        def span_body(sp, carry):
            s0 = (2 * sp + c) * 128
            for g in range(8):
                run_v[pl.ds(g * 16, 16)] = one

            def wbody(w, carry2):
                t0 = 2 * w

                @pl.when((sp > 0) | (w > 0))
                def _():
                    dma_drain(ob0, sem0)

                compute_chunk(t0, s0, ob0)
                dma_start(ob0, t0, s0, sem0)

                @pl.when((sp > 0) | (w > 0))
                def _():
                    dma_drain(ob1, sem1)

                compute_chunk(t0 + 1, s0, ob1)
                dma_start(ob1, t0 + 1, s0, sem1)
                return carry2

            lax.fori_loop(0, 2, wbody, jnp.int32(0))
            return carry

        lax.fori_loop(0, 4, span_body, jnp.int32(0))
        dma_drain(ob0, sem0)
        dma_drain(ob1, sem1)

    @pl.when(s < 8)
    def _():
        run_direction(fwd_hbm, rev=False)

    @pl.when(s >= 8)
    def _():
        run_direction(bwd_hbm, rev=True)


@functools.partial(jax.jit, static_argnames=())
def _gate_net(score, score_idx):
    mesh = plsc.VectorSubcoreMesh(core_axis_name="c", subcore_axis_name="s")
    out_ty = (jax.ShapeDtypeStruct((B, ROWS, M), jnp.float32),
              jax.ShapeDtypeStruct((B, ROWS, M), jnp.float32))
    fn = pl.kernel(
        _body,
        mesh=mesh,
        out_type=out_ty,
        scratch_types=[
            pltpu.VMEM((B * L,), jnp.float32),
            pltpu.VMEM((L,), jnp.int32),
            pltpu.VMEM((8 + L + 8,), jnp.float32),
            pltpu.VMEM((8 + L + 8,), jnp.float32),
            pltpu.VMEM((8 + L + 8,), jnp.float32),
            pltpu.VMEM((2 * L + 32,), jnp.float32),
            pltpu.VMEM((L + 16,), jnp.float32),
            pltpu.VMEM((128,), jnp.float32),
            pltpu.VMEM((256, 128), jnp.float32),
            pltpu.VMEM((256, 128), jnp.float32),
            pltpu.SemaphoreType.DMA,
            pltpu.SemaphoreType.DMA,
        ],
        compiler_params=pltpu.CompilerParams(needs_layout_passes=False),
    )
    return fn(score, score_idx)


def kernel(score, rep_srcs, rep_idx, score_idx):
    del rep_srcs, rep_idx
    return _gate_net(score, score_idx.astype(jnp.int32))
